# baseline (speedup 1.0000x reference)
import functools

import jax
import jax.numpy as jnp
from jax.experimental import pallas as pl

_FRAMEPOINTS = (
    (1.0, 1.0, 1.0), (1.0, 1.0, -1.0), (1.0, -1.0, 1.0), (1.0, -1.0, -1.0),
    (-1.0, 1.0, 1.0), (-1.0, 1.0, -1.0), (-1.0, -1.0, 1.0), (-1.0, -1.0, -1.0),
    (0.0, 0.0, 0.0),
)


def _t(x):
    return jnp.transpose(x, (0, 2, 1))


def _sqdist(a, b):
    return (jnp.sum(a * a, -1)[..., None] + jnp.sum(b * b, -1)[:, None, :]
            - 2.0 * jnp.einsum('bnc,bsc->bns', a, b))


def _gather(points, idx):
    return jax.vmap(lambda p, i: p[i])(points, idx)


def _fps(xyz, npoint):
    B, N, _ = xyz.shape

    def body(i, state):
        dists, farthest, idxs = state
        idxs = idxs.at[:, i].set(farthest)
        centroid = jax.vmap(lambda p, j: p[j])(xyz, farthest[:, None])
        d = jnp.sum((xyz - centroid) ** 2, -1)
        dists = jnp.minimum(dists, d)
        farthest = jnp.argmax(dists, -1).astype(jnp.int32)
        return dists, farthest, idxs

    dists = jnp.full((B, N), 1e10, jnp.float32)
    farthest = jnp.zeros((B,), jnp.int32)
    idxs = jnp.zeros((B, npoint), jnp.int32)
    _, _, idxs = jax.lax.fori_loop(0, npoint, body, (dists, farthest, idxs))
    return idxs


def _ball_query(xyz, new_xyz, radius, nsample):
    d = _sqdist(new_xyz, xyz)
    neg, idx = jax.lax.top_k(-d, nsample)
    dk = -neg
    idx = jnp.where(dk > radius * radius, idx[..., :1], idx)
    return idx


def _fpac_body(rel_ref, gf_ref, fpT_ref, fpsq_ref, w1_ref, b1_ref, w2_ref,
               b2_ref, w3_ref, b3_ref, pw_ref, pb_ref, out_ref, *, inv_r):
    _, Ts, K, _ = rel_ref.shape
    n = Ts * K
    cin = gf_ref.shape[-1]
    cout = out_ref.shape[-1]
    rel = rel_ref[0].reshape(n, 3) * inv_r
    sq = jnp.sum(rel * rel, axis=1, keepdims=True)
    dot = jnp.dot(rel, fpT_ref[...], preferred_element_type=jnp.float32)
    rbf = jnp.exp(-(sq + fpsq_ref[...] - 2.0 * dot))
    t1 = jnp.maximum(
        jnp.dot(rbf, w1_ref[...], preferred_element_type=jnp.float32)
        + b1_ref[...], 0.0)
    t2 = jnp.maximum(t1 * w2_ref[...] + b2_ref[...], 0.0)
    w = jnp.dot(t2, w3_ref[...], preferred_element_type=jnp.float32) + b3_ref[...]
    f = (jnp.dot(gf_ref[0].reshape(n, cin), pw_ref[...],
                 preferred_element_type=jnp.float32) + pb_ref[...])
    o = jnp.maximum(f * w, 0.0).reshape(Ts, K, cout)
    out_ref[0] = jnp.max(o, axis=1)


def _fpac_pallas(rel, gf, p, radius):
    B, S, K, _ = rel.shape
    cin = gf.shape[-1]
    h2 = p['w2']['W'].shape[1]
    cout = p['w3']['W'].shape[1]
    Ts = max(1, min(S, 2048 // K))
    while S % Ts:
        Ts //= 2
    fpT = jnp.array(_FRAMEPOINTS, jnp.float32).T
    fpsq = jnp.sum(fpT * fpT, axis=0)[None, :]
    args = (
        rel, gf, fpT, fpsq,
        p['w1']['W'], p['w1']['b'][None, :],
        p['w2']['W'], p['w2']['b'][None, :],
        p['w3']['W'], p['w3']['b'][None, :],
        p['proj']['W'], p['proj']['b'][None, :],
    )
    full = lambda a: pl.BlockSpec(a.shape, lambda b, t: (0,) * a.ndim)
    in_specs = [
        pl.BlockSpec((1, Ts, K, 3), lambda b, t: (b, t, 0, 0)),
        pl.BlockSpec((1, Ts, K, cin), lambda b, t: (b, t, 0, 0)),
    ] + [full(a) for a in args[2:]]
    out = pl.pallas_call(
        functools.partial(_fpac_body, inv_r=1.0 / radius),
        grid=(B, S // Ts),
        in_specs=in_specs,
        out_specs=pl.BlockSpec((1, Ts, cout), lambda b, t: (b, t, 0)),
        out_shape=jax.ShapeDtypeStruct((B, S, cout), jnp.float32),
    )(*args)
    return out


def _fpac(xyz, feats, p, npoint, radius, nsample):
    B, N, _ = xyz.shape
    if npoint is None:
        new_xyz = jnp.zeros((B, 1, 3), jnp.float32)
        rel = xyz[:, None, :, :]
        g_feat = feats[:, None, :, :]
    else:
        idx = _fps(xyz, npoint)
        new_xyz = _gather(xyz, idx)
        gi = _ball_query(xyz, new_xyz, radius, nsample)
        g_xyz = _gather(xyz, gi)
        g_feat = _gather(feats, gi)
        rel = g_xyz - new_xyz[:, :, None, :]
    out = _fpac_pallas(rel, g_feat, p, radius)
    return out, new_xyz


def _conv1d(x, d):
    return jnp.einsum('bcn,cd->bdn', x, d['W']) + d['b'][None, :, None]


def _bn_relu(x, bn):
    mu = jnp.mean(x, axis=(0, 2), keepdims=True)
    var = jnp.var(x, axis=(0, 2), keepdims=True)
    xh = (x - mu) / jnp.sqrt(var + 1e-5)
    return jax.nn.relu(xh * bn['gamma'][None, :, None] + bn['beta'][None, :, None])


def _fp(xyz1, xyz2, points1, points2, p):
    N = xyz1.shape[2]
    S = xyz2.shape[2]
    if S == 1:
        interp = jnp.repeat(points2, N, axis=2)
    else:
        d = _sqdist(_t(xyz1), _t(xyz2))
        negd, idx = jax.lax.top_k(-d, 3)
        dist = jnp.maximum(-negd, 1e-10)
        w = 1.0 / (dist + 1e-8)
        w = w / jnp.sum(w, -1, keepdims=True)
        g = _gather(_t(points2), idx)
        interp = _t(jnp.sum(g * w[..., None], axis=2))
    new = interp if points1 is None else jnp.concatenate([points1, interp], axis=1)
    for conv, bn in zip(p['convs'], p['bns']):
        new = _bn_relu(_conv1d(new, conv), bn)
    return new


def kernel(xyz, cls_label, params):
    B, C, N = xyz.shape
    xyzT = _t(xyz)
    f1, s1 = _fpac(xyzT, xyzT, params['fpac1'], 512, 0.2, 32)
    f2, s2 = _fpac(s1, f1, params['fpac2'], 128, 0.4, 64)
    f3, s3 = _fpac(s2, f2, params['fpac3'], None, 0.8, 32)
    l2 = _fp(_t(s2), _t(s3), _t(f2), _t(f3), params['fp3'])
    l1 = _fp(_t(s1), _t(s2), _t(f1), l2, params['fp2'])
    cls_oh = jnp.repeat(cls_label[:, :, None], N, axis=2)
    l0_in = jnp.concatenate([cls_oh, xyz, xyz], axis=1)
    l0 = _fp(xyz, _t(s1), l0_in, l1, params['fp1'])
    feat = _bn_relu(_conv1d(l0, params['conv1']), params['bn1'])
    x = _conv1d(feat, params['conv2'])
    x = jax.nn.log_softmax(x, axis=1)
    return _t(x), _t(f3)


# speedup vs baseline: 1.3297x; 1.3297x over previous
import functools

import jax
import jax.numpy as jnp
from jax.experimental import pallas as pl

_FRAMEPOINTS = (
    (1.0, 1.0, 1.0), (1.0, 1.0, -1.0), (1.0, -1.0, 1.0), (1.0, -1.0, -1.0),
    (-1.0, 1.0, 1.0), (-1.0, 1.0, -1.0), (-1.0, -1.0, 1.0), (-1.0, -1.0, -1.0),
    (0.0, 0.0, 0.0),
)


def _t(x):
    return jnp.transpose(x, (0, 2, 1))


def _sqdist(a, b):
    return (jnp.sum(a * a, -1)[..., None] + jnp.sum(b * b, -1)[:, None, :]
            - 2.0 * jnp.einsum('bnc,bsc->bns', a, b))


def _gather(points, idx):
    return jax.vmap(lambda p, i: p[i])(points, idx)


def _fps_body(x_ref, o_ref, *, npoint):
    x, y, z = x_ref[0], x_ref[1], x_ref[2]
    B, N = x.shape
    lane = jax.lax.broadcasted_iota(jnp.int32, (B, N), 1).astype(jnp.float32)
    ocol = jax.lax.broadcasted_iota(jnp.int32, (B, npoint), 1).astype(jnp.float32)

    def body(i, st):
        dists, far, acc = st
        onehot = (ocol == i.astype(jnp.float32)).astype(jnp.float32)
        acc = acc * (1.0 - onehot) + far * onehot
        mask = lane == far
        cx = jnp.sum(jnp.where(mask, x, 0.0), 1, keepdims=True)
        cy = jnp.sum(jnp.where(mask, y, 0.0), 1, keepdims=True)
        cz = jnp.sum(jnp.where(mask, z, 0.0), 1, keepdims=True)
        d = (x - cx) ** 2 + (y - cy) ** 2 + (z - cz) ** 2
        dists = jnp.minimum(dists, d)
        m = jnp.max(dists, 1, keepdims=True)
        far = jnp.min(jnp.where(dists == m, lane, float(N)), 1, keepdims=True)
        return dists, far, acc

    dists = x * 0.0 + 1e10
    far = jnp.min(x * 0.0, 1, keepdims=True)
    acc = ocol + x[:, :npoint] * 0.0
    _, _, acc = jax.lax.fori_loop(0, npoint, body, (dists, far, acc))
    o_ref[...] = acc.astype(jnp.int32)


def _fps(xyz, npoint):
    B, N, _ = xyz.shape
    x3 = jnp.transpose(xyz, (2, 0, 1))
    return pl.pallas_call(
        functools.partial(_fps_body, npoint=npoint),
        out_shape=jax.ShapeDtypeStruct((B, npoint), jnp.int32),
    )(x3)


def _ball_query(xyz, new_xyz, radius, nsample):
    d = _sqdist(new_xyz, xyz)
    neg, idx = jax.lax.top_k(-d, nsample)
    dk = -neg
    idx = jnp.where(dk > radius * radius, idx[..., :1], idx)
    return idx


def _fpac_body(rel_ref, gf_ref, fpT_ref, fpsq_ref, w1_ref, b1_ref, w2_ref,
               b2_ref, w3_ref, b3_ref, pw_ref, pb_ref, out_ref, *, inv_r):
    _, Ts, K, _ = rel_ref.shape
    n = Ts * K
    cin = gf_ref.shape[-1]
    cout = out_ref.shape[-1]
    rel = rel_ref[0].reshape(n, 3) * inv_r
    sq = jnp.sum(rel * rel, axis=1, keepdims=True)
    dot = jnp.dot(rel, fpT_ref[...], preferred_element_type=jnp.float32)
    rbf = jnp.exp(-(sq + fpsq_ref[...] - 2.0 * dot))
    t1 = jnp.maximum(
        jnp.dot(rbf, w1_ref[...], preferred_element_type=jnp.float32)
        + b1_ref[...], 0.0)
    t2 = jnp.maximum(t1 * w2_ref[...] + b2_ref[...], 0.0)
    w = jnp.dot(t2, w3_ref[...], preferred_element_type=jnp.float32) + b3_ref[...]
    f = (jnp.dot(gf_ref[0].reshape(n, cin), pw_ref[...],
                 preferred_element_type=jnp.float32) + pb_ref[...])
    o = jnp.maximum(f * w, 0.0).reshape(Ts, K, cout)
    out_ref[0] = jnp.max(o, axis=1)


def _fpac_pallas(rel, gf, p, radius):
    B, S, K, _ = rel.shape
    cin = gf.shape[-1]
    h2 = p['w2']['W'].shape[1]
    cout = p['w3']['W'].shape[1]
    Ts = max(1, min(S, 2048 // K))
    while S % Ts:
        Ts //= 2
    fpT = jnp.array(_FRAMEPOINTS, jnp.float32).T
    fpsq = jnp.sum(fpT * fpT, axis=0)[None, :]
    args = (
        rel, gf, fpT, fpsq,
        p['w1']['W'], p['w1']['b'][None, :],
        p['w2']['W'], p['w2']['b'][None, :],
        p['w3']['W'], p['w3']['b'][None, :],
        p['proj']['W'], p['proj']['b'][None, :],
    )
    full = lambda a: pl.BlockSpec(a.shape, lambda b, t: (0,) * a.ndim)
    in_specs = [
        pl.BlockSpec((1, Ts, K, 3), lambda b, t: (b, t, 0, 0)),
        pl.BlockSpec((1, Ts, K, cin), lambda b, t: (b, t, 0, 0)),
    ] + [full(a) for a in args[2:]]
    out = pl.pallas_call(
        functools.partial(_fpac_body, inv_r=1.0 / radius),
        grid=(B, S // Ts),
        in_specs=in_specs,
        out_specs=pl.BlockSpec((1, Ts, cout), lambda b, t: (b, t, 0)),
        out_shape=jax.ShapeDtypeStruct((B, S, cout), jnp.float32),
    )(*args)
    return out


def _fpac(xyz, feats, p, npoint, radius, nsample):
    B, N, _ = xyz.shape
    if npoint is None:
        new_xyz = jnp.zeros((B, 1, 3), jnp.float32)
        rel = xyz[:, None, :, :]
        g_feat = feats[:, None, :, :]
    else:
        idx = _fps(xyz, npoint)
        new_xyz = _gather(xyz, idx)
        gi = _ball_query(xyz, new_xyz, radius, nsample)
        g_xyz = _gather(xyz, gi)
        g_feat = _gather(feats, gi)
        rel = g_xyz - new_xyz[:, :, None, :]
    out = _fpac_pallas(rel, g_feat, p, radius)
    return out, new_xyz


def _conv1d(x, d):
    return jnp.einsum('bcn,cd->bdn', x, d['W']) + d['b'][None, :, None]


def _bn_relu(x, bn):
    mu = jnp.mean(x, axis=(0, 2), keepdims=True)
    var = jnp.var(x, axis=(0, 2), keepdims=True)
    xh = (x - mu) / jnp.sqrt(var + 1e-5)
    return jax.nn.relu(xh * bn['gamma'][None, :, None] + bn['beta'][None, :, None])


def _fp(xyz1, xyz2, points1, points2, p):
    N = xyz1.shape[2]
    S = xyz2.shape[2]
    if S == 1:
        interp = jnp.repeat(points2, N, axis=2)
    else:
        d = _sqdist(_t(xyz1), _t(xyz2))
        negd, idx = jax.lax.top_k(-d, 3)
        dist = jnp.maximum(-negd, 1e-10)
        w = 1.0 / (dist + 1e-8)
        w = w / jnp.sum(w, -1, keepdims=True)
        g = _gather(_t(points2), idx)
        interp = _t(jnp.sum(g * w[..., None], axis=2))
    new = interp if points1 is None else jnp.concatenate([points1, interp], axis=1)
    for conv, bn in zip(p['convs'], p['bns']):
        new = _bn_relu(_conv1d(new, conv), bn)
    return new


def kernel(xyz, cls_label, params):
    B, C, N = xyz.shape
    xyzT = _t(xyz)
    f1, s1 = _fpac(xyzT, xyzT, params['fpac1'], 512, 0.2, 32)
    f2, s2 = _fpac(s1, f1, params['fpac2'], 128, 0.4, 64)
    f3, s3 = _fpac(s2, f2, params['fpac3'], None, 0.8, 32)
    l2 = _fp(_t(s2), _t(s3), _t(f2), _t(f3), params['fp3'])
    l1 = _fp(_t(s1), _t(s2), _t(f1), l2, params['fp2'])
    cls_oh = jnp.repeat(cls_label[:, :, None], N, axis=2)
    l0_in = jnp.concatenate([cls_oh, xyz, xyz], axis=1)
    l0 = _fp(xyz, _t(s1), l0_in, l1, params['fp1'])
    feat = _bn_relu(_conv1d(l0, params['conv1']), params['bn1'])
    x = _conv1d(feat, params['conv2'])
    x = jax.nn.log_softmax(x, axis=1)
    return _t(x), _t(f3)


# speedup vs baseline: 1.3299x; 1.0001x over previous
import functools

import jax
import jax.numpy as jnp
from jax.experimental import pallas as pl

_FRAMEPOINTS = (
    (1.0, 1.0, 1.0), (1.0, 1.0, -1.0), (1.0, -1.0, 1.0), (1.0, -1.0, -1.0),
    (-1.0, 1.0, 1.0), (-1.0, 1.0, -1.0), (-1.0, -1.0, 1.0), (-1.0, -1.0, -1.0),
    (0.0, 0.0, 0.0),
)


def _t(x):
    return jnp.transpose(x, (0, 2, 1))


def _sqdist(a, b):
    return (jnp.sum(a * a, -1)[..., None] + jnp.sum(b * b, -1)[:, None, :]
            - 2.0 * jnp.einsum('bnc,bsc->bns', a, b))


def _gather(points, idx):
    return jax.vmap(lambda p, i: p[i])(points, idx)


def _fps_body(x_ref, o_ref, *, npoint):
    x, y, z = x_ref[0], x_ref[1], x_ref[2]
    B, N = x.shape
    lane = jax.lax.broadcasted_iota(jnp.int32, (B, N), 1).astype(jnp.float32)
    ocol = jax.lax.broadcasted_iota(jnp.int32, (B, npoint), 1).astype(jnp.float32)

    def body(i, st):
        dists, far, acc = st
        onehot = (ocol == i.astype(jnp.float32)).astype(jnp.float32)
        acc = acc * (1.0 - onehot) + far * onehot
        mask = lane == far
        cx = jnp.sum(jnp.where(mask, x, 0.0), 1, keepdims=True)
        cy = jnp.sum(jnp.where(mask, y, 0.0), 1, keepdims=True)
        cz = jnp.sum(jnp.where(mask, z, 0.0), 1, keepdims=True)
        d = (x - cx) ** 2 + (y - cy) ** 2 + (z - cz) ** 2
        dists = jnp.minimum(dists, d)
        m = jnp.max(dists, 1, keepdims=True)
        far = jnp.min(jnp.where(dists == m, lane, float(N)), 1, keepdims=True)
        return dists, far, acc

    dists = x * 0.0 + 1e10
    far = jnp.min(x * 0.0, 1, keepdims=True)
    acc = ocol + x[:, :npoint] * 0.0
    _, _, acc = jax.lax.fori_loop(0, npoint, body, (dists, far, acc))
    o_ref[...] = acc.astype(jnp.int32)


def _fps(xyz, npoint):
    B, N, _ = xyz.shape
    x3 = jnp.transpose(xyz, (2, 0, 1))
    return pl.pallas_call(
        functools.partial(_fps_body, npoint=npoint),
        out_shape=jax.ShapeDtypeStruct((B, npoint), jnp.int32),
    )(x3)


def _ball_query(xyz, new_xyz, radius, nsample):
    d = _sqdist(new_xyz, xyz)
    neg, idx = jax.lax.top_k(-d, nsample)
    dk = -neg
    idx = jnp.where(dk > radius * radius, idx[..., :1], idx)
    return idx


def _fpac_body(rel_ref, gf_ref, fpT_ref, fpsq_ref, w1_ref, b1_ref, w2_ref,
               b2_ref, w3_ref, b3_ref, pw_ref, pb_ref, out_ref, *, inv_r):
    _, Ts, K, _ = rel_ref.shape
    n = Ts * K
    cin = gf_ref.shape[-1]
    cout = out_ref.shape[-1]
    rel = rel_ref[0].reshape(n, 3) * inv_r
    sq = jnp.sum(rel * rel, axis=1, keepdims=True)
    dot = jnp.dot(rel, fpT_ref[...], preferred_element_type=jnp.float32)
    rbf = jnp.exp(-(sq + fpsq_ref[...] - 2.0 * dot))
    t1 = jnp.maximum(
        jnp.dot(rbf, w1_ref[...], preferred_element_type=jnp.float32)
        + b1_ref[...], 0.0)
    t2 = jnp.maximum(t1 * w2_ref[...] + b2_ref[...], 0.0)
    w = jnp.dot(t2, w3_ref[...], preferred_element_type=jnp.float32) + b3_ref[...]
    f = (jnp.dot(gf_ref[0].reshape(n, cin), pw_ref[...],
                 preferred_element_type=jnp.float32) + pb_ref[...])
    o = jnp.maximum(f * w, 0.0).reshape(Ts, K, cout)
    out_ref[0] = jnp.max(o, axis=1)


def _fpac_pallas(rel, gf, p, radius):
    B, S, K, _ = rel.shape
    cin = gf.shape[-1]
    h2 = p['w2']['W'].shape[1]
    cout = p['w3']['W'].shape[1]
    Ts = max(1, min(S, 2048 // K))
    while S % Ts:
        Ts //= 2
    fpT = jnp.array(_FRAMEPOINTS, jnp.float32).T
    fpsq = jnp.sum(fpT * fpT, axis=0)[None, :]
    args = (
        rel, gf, fpT, fpsq,
        p['w1']['W'], p['w1']['b'][None, :],
        p['w2']['W'], p['w2']['b'][None, :],
        p['w3']['W'], p['w3']['b'][None, :],
        p['proj']['W'], p['proj']['b'][None, :],
    )
    full = lambda a: pl.BlockSpec(a.shape, lambda b, t: (0,) * a.ndim)
    in_specs = [
        pl.BlockSpec((1, Ts, K, 3), lambda b, t: (b, t, 0, 0)),
        pl.BlockSpec((1, Ts, K, cin), lambda b, t: (b, t, 0, 0)),
    ] + [full(a) for a in args[2:]]
    out = pl.pallas_call(
        functools.partial(_fpac_body, inv_r=1.0 / radius),
        grid=(B, S // Ts),
        in_specs=in_specs,
        out_specs=pl.BlockSpec((1, Ts, cout), lambda b, t: (b, t, 0)),
        out_shape=jax.ShapeDtypeStruct((B, S, cout), jnp.float32),
    )(*args)
    return out


def _fpac(xyz, feats, p, npoint, radius, nsample):
    B, N, _ = xyz.shape
    if npoint is None:
        new_xyz = jnp.zeros((B, 1, 3), jnp.float32)
        rel = xyz[:, None, :, :]
        g_feat = feats[:, None, :, :]
    else:
        idx = _fps(xyz, npoint)
        new_xyz = _gather(xyz, idx)
        gi = _ball_query(xyz, new_xyz, radius, nsample)
        g_xyz = _gather(xyz, gi)
        g_feat = _gather(feats, gi)
        rel = g_xyz - new_xyz[:, :, None, :]
    out = _fpac_pallas(rel, g_feat, p, radius)
    return out, new_xyz


def _conv1d(x, d):
    return jnp.einsum('bcn,cd->bdn', x, d['W']) + d['b'][None, :, None]


def _bn_relu(x, bn):
    mu = jnp.mean(x, axis=(0, 2), keepdims=True)
    var = jnp.var(x, axis=(0, 2), keepdims=True)
    xh = (x - mu) / jnp.sqrt(var + 1e-5)
    return jax.nn.relu(xh * bn['gamma'][None, :, None] + bn['beta'][None, :, None])


def _interp_body(a_ref, bT_ref, p2_ref, o_ref):
    _, Ts, _ = a_ref.shape
    S = bT_ref.shape[2]
    a = a_ref[0]
    bT = bT_ref[0]
    asq = jnp.sum(a * a, 1, keepdims=True)
    bsq = jnp.sum(bT * bT, 0, keepdims=True)
    d = asq + bsq - 2.0 * jnp.dot(a, bT, preferred_element_type=jnp.float32,
                                  precision=jax.lax.Precision.HIGHEST)
    lane = jax.lax.broadcasted_iota(jnp.int32, (Ts, S), 1).astype(jnp.float32)
    ws, ihots = [], []
    for _k in range(3):
        m = jnp.min(d, 1, keepdims=True)
        ik = jnp.min(jnp.where(d == m, lane, float(S)), 1, keepdims=True)
        ws.append(1.0 / (jnp.maximum(m, 1e-10) + 1e-8))
        hot = (lane == ik).astype(jnp.float32)
        ihots.append(hot)
        d = d + hot * 1e30
    wsum = ws[0] + ws[1] + ws[2]
    Wm = sum((w / wsum) * hot for w, hot in zip(ws, ihots))
    o_ref[0] = jnp.dot(Wm, p2_ref[0], preferred_element_type=jnp.float32)


def _interp_pallas(a, bT, p2):
    B, N, _ = a.shape
    S = bT.shape[2]
    C = p2.shape[2]
    Ts = 128
    return pl.pallas_call(
        _interp_body,
        grid=(B, N // Ts),
        in_specs=[
            pl.BlockSpec((1, Ts, 3), lambda b, t: (b, t, 0)),
            pl.BlockSpec((1, 3, S), lambda b, t: (b, 0, 0)),
            pl.BlockSpec((1, S, C), lambda b, t: (b, 0, 0)),
        ],
        out_specs=pl.BlockSpec((1, Ts, C), lambda b, t: (b, t, 0)),
        out_shape=jax.ShapeDtypeStruct((B, N, C), jnp.float32),
    )(a, bT, p2)


def _fp(xyz1, xyz2, points1, points2, p):
    N = xyz1.shape[2]
    S = xyz2.shape[2]
    if S == 1:
        interp = jnp.repeat(points2, N, axis=2)
    else:
        d = _sqdist(_t(xyz1), _t(xyz2))
        negd, idx = jax.lax.top_k(-d, 3)
        dist = jnp.maximum(-negd, 1e-10)
        w = 1.0 / (dist + 1e-8)
        w = w / jnp.sum(w, -1, keepdims=True)
        g = _gather(_t(points2), idx)
        interp = _t(jnp.sum(g * w[..., None], axis=2))
    new = interp if points1 is None else jnp.concatenate([points1, interp], axis=1)
    for conv, bn in zip(p['convs'], p['bns']):
        new = _bn_relu(_conv1d(new, conv), bn)
    return new


def kernel(xyz, cls_label, params):
    B, C, N = xyz.shape
    xyzT = _t(xyz)
    f1, s1 = _fpac(xyzT, xyzT, params['fpac1'], 512, 0.2, 32)
    f2, s2 = _fpac(s1, f1, params['fpac2'], 128, 0.4, 64)
    f3, s3 = _fpac(s2, f2, params['fpac3'], None, 0.8, 32)
    l2 = _fp(_t(s2), _t(s3), _t(f2), _t(f3), params['fp3'])
    l1 = _fp(_t(s1), _t(s2), _t(f1), l2, params['fp2'])
    cls_oh = jnp.repeat(cls_label[:, :, None], N, axis=2)
    l0_in = jnp.concatenate([cls_oh, xyz, xyz], axis=1)
    l0 = _fp(xyz, _t(s1), l0_in, l1, params['fp1'])
    feat = _bn_relu(_conv1d(l0, params['conv1']), params['bn1'])
    x = _conv1d(feat, params['conv2'])
    x = jax.nn.log_softmax(x, axis=1)
    return _t(x), _t(f3)
